# baseline (speedup 1.0000x reference)
import jax
import jax.numpy as jnp
from jax import lax
from jax.experimental import pallas as pl
from jax.experimental.pallas import tpu as pltpu
from jax.experimental.pallas import tpu_sc as plsc

N = 10000
E = 320000
D = 128
OUT = 64

NC = 2
NS = 16
G = 128
ITERS = 80
CH = 8
NCH = ITERS // CH
EPAD = NC * NS * ITERS * G
NACC = N + 8
ROWS_SUB = 640
ROWS_LAST = N - ROWS_SUB * (NS - 1)

BN = 1000
f32 = jnp.float32


def _zero_fill(zbuf):
    z16 = jnp.zeros((16,), f32)
    for r in range(16):
        for c in range(D // 16):
            zbuf[r, pl.ds(c * 16, 16)] = z16


def _acc_chunks(sid):
    nz = jnp.where(sid < NS - 1, ROWS_SUB // 16, ROWS_LAST // 16)
    nd = jnp.where(sid < NS - 1, ROWS_SUB // 80, ROWS_LAST // 80)
    return sid * ROWS_SUB, nz, nd


def _segsum_body(table, src2, dst2, s_out,
                 srcc0, srcc1, dstc0, dstc1, rows0, rows1, zbuf, acc,
                 gs0, gs1, ss0, ss1, is0, is1, id0, id1):
    cid = lax.axis_index("c")
    sid = lax.axis_index("s")
    wid = cid * NS + sid
    row0, nz, nd = _acc_chunks(sid)

    srcc = (srcc0, srcc1)
    dstc = (dstc0, dstc1)
    rows = (rows0, rows1)
    gsem = (gs0, gs1)
    ssem = (ss0, ss1)
    isem = (is0, is1)
    idem = (id0, id1)

    def iload(c, p):
        pltpu.async_copy(src2.at[wid, pl.ds(c * CH, CH)], srcc[p], isem[p])
        pltpu.async_copy(dst2.at[wid, pl.ds(c * CH, CH)], dstc[p], idem[p])

    def wait_i(p):
        pltpu.make_async_copy(src2.at[wid, pl.ds(0, CH)], srcc[p],
                              isem[p]).wait()
        pltpu.make_async_copy(dst2.at[wid, pl.ds(0, CH)], dstc[p],
                              idem[p]).wait()

    def gat(p, j, b):
        return pltpu.async_copy(table.at[srcc[p].at[j]], rows[b], gsem[b])

    def sca(p, j, b):
        return pltpu.async_copy(rows[b], acc.at[dstc[p].at[j]], ssem[b],
                                add=True)

    def wait_g(b):
        pltpu.make_async_copy(table.at[srcc[0].at[0]], rows[b],
                              gsem[b]).wait()

    def wait_s(b):
        pltpu.make_async_copy(rows[b], acc.at[dstc[0].at[0]], ssem[b]).wait()

    iload(0, 0)
    iload(1, 1)

    _zero_fill(zbuf)
    lax.fori_loop(
        0, nz,
        lambda k, _: (pltpu.sync_copy(zbuf, acc.at[pl.ds(row0 + k * 16, 16)]),
                      0)[1], 0)
    plsc.subcore_barrier()

    wait_i(0)
    gat(0, 0, 0)

    def two_chunks(m, _):
        for half in range(2):
            c = 2 * m + half
            for j in range(CH):
                i = c * CH + j
                b = j % 2
                wait_g(b)
                sca(half, j, b)

                @pl.when((i > 0) & (i < ITERS - 1))
                def _():
                    wait_s(1 - b)
                if j == 0:
                    @pl.when((c > 0) & (c + 1 < NCH))
                    def _():
                        iload(c + 1, 1 - half)
                if j < CH - 1:
                    @pl.when(i < ITERS - 1)
                    def _():
                        gat(half, j + 1, 1 - b)
                else:
                    @pl.when(i < ITERS - 1)
                    def _():
                        wait_i(1 - half)
                        gat(1 - half, 0, 1 - b)
        return 0
    lax.fori_loop(0, NCH // 2, two_chunks, 0)
    wait_s(0)
    wait_s(1)

    plsc.subcore_barrier()

    def drain(k, _):
        b = row0 + k * 80
        pltpu.sync_copy(acc.at[pl.ds(b, 80)], s_out.at[cid, pl.ds(b, 80)])
        return 0
    lax.fori_loop(0, nd, drain, 0)


def _deg_body(dst2, deg_out, dstl, ones_v, zbuf, acc, ss0, ss1):
    cid = lax.axis_index("c")
    sid = lax.axis_index("s")
    wid = cid * NS + sid
    row0, nz, nd = _acc_chunks(sid)

    pltpu.sync_copy(dst2.at[wid], dstl)

    _zero_fill(zbuf)
    one16 = jnp.ones((16,), f32)
    for r in range(G):
        for c in range(D // 16):
            ones_v[r, pl.ds(c * 16, 16)] = one16
    lax.fori_loop(
        0, nz,
        lambda k, _: (pltpu.sync_copy(zbuf, acc.at[pl.ds(row0 + k * 16, 16)]),
                      0)[1], 0)
    plsc.subcore_barrier()

    ssem = (ss0, ss1)

    def sca(i, b):
        return pltpu.async_copy(ones_v, acc.at[dstl.at[i]], ssem[b],
                                add=True)

    def wait_s(b):
        pltpu.make_async_copy(ones_v, acc.at[dstl.at[0]], ssem[b]).wait()

    sca(0, 0)
    sca(1, 1)

    def pair(o, _):
        wait_s(0)
        sca(2 * o + 2, 0)
        wait_s(1)
        sca(2 * o + 3, 1)
        return 0
    lax.fori_loop(0, ITERS // 2 - 1, pair, 0)
    wait_s(0)
    wait_s(1)

    plsc.subcore_barrier()

    def drain(k, _):
        b = row0 + k * 80
        pltpu.sync_copy(acc.at[pl.ds(b, 80)], deg_out.at[cid, pl.ds(b, 80)])
        return 0
    lax.fori_loop(0, nd, drain, 0)


_MESH = plsc.VectorSubcoreMesh(core_axis_name="c", subcore_axis_name="s",
                               num_cores=NC, num_subcores=NS)

_segsum = pl.kernel(
    _segsum_body,
    out_type=jax.ShapeDtypeStruct((NC, N, D), f32),
    mesh=_MESH,
    scratch_types=(pltpu.VMEM((CH, G), jnp.int32),
                   pltpu.VMEM((CH, G), jnp.int32),
                   pltpu.VMEM((CH, G), jnp.int32),
                   pltpu.VMEM((CH, G), jnp.int32),
                   pltpu.VMEM((G, D), f32),
                   pltpu.VMEM((G, D), f32),
                   pltpu.VMEM((16, D), f32),
                   pltpu.VMEM_SHARED((NACC, D), f32))
    + (pltpu.SemaphoreType.DMA,) * 8)

_deg_count = pl.kernel(
    _deg_body,
    out_type=jax.ShapeDtypeStruct((NC, N, D), f32),
    mesh=_MESH,
    scratch_types=(pltpu.VMEM((ITERS, G), jnp.int32),
                   pltpu.VMEM((G, D), f32),
                   pltpu.VMEM((16, D), f32),
                   pltpu.VMEM_SHARED((NACC, D), f32),
                   pltpu.SemaphoreType.DMA,
                   pltpu.SemaphoreType.DMA))


def _stage_a_body(tau_ref, x_ref, sxp_ref, degp_ref, h_ref, dinv_ref):
    deg = jnp.maximum(degp_ref[0, :, 0] + degp_ref[1, :, 0], 1.0)
    dinv = (1.0 / deg)[:, None]
    s = sxp_ref[0] + sxp_ref[1]
    xb = x_ref[...]
    h_ref[...] = xb + tau_ref[0, 0] * (s * dinv - xb)
    dinv_ref[...] = jnp.broadcast_to(dinv, (dinv.shape[0], 8))


def _stage_bc_body(h_ref, sp_ref, dinv_ref, w_ref, g_ref):
    dinv = dinv_ref[:, 0][:, None]
    msg = (sp_ref[0] + sp_ref[1]) * dinv - h_ref[...]
    g_ref[...] = jnp.tanh(jnp.dot(msg, w_ref[...],
                                  precision=lax.Precision.HIGHEST))


def _stage_d_body(g1_ref, g2_ref, s1p_ref, s2p_ref, dinv_ref,
                  wconv_ref, wmlp_ref, out_ref):
    dinv = dinv_ref[:, 0][:, None]
    cat = jnp.concatenate(
        [g1_ref[...], g2_ref[...],
         (s1p_ref[0] + s1p_ref[1]) * dinv,
         (s2p_ref[0] + s2p_ref[1]) * dinv], axis=1)
    pre = jax.nn.relu(jnp.dot(cat, wconv_ref[...],
                              precision=lax.Precision.HIGHEST))
    out_ref[...] = jnp.dot(pre, wmlp_ref[...],
                           precision=lax.Precision.HIGHEST)


def _blk():
    return pl.BlockSpec((BN, D), lambda i: (i, 0))


def _pblk():
    return pl.BlockSpec((NC, BN, D), lambda i: (0, i, 0))


def _dinv_blk():
    return pl.BlockSpec((BN, 8), lambda i: (i, 0))


def _full(shape):
    return pl.BlockSpec(shape, lambda i: (0, 0))


_GRID = (N // BN,)


def _stage_a(x, sxp, degp, tau2):
    return pl.pallas_call(
        _stage_a_body,
        grid=_GRID,
        in_specs=[pl.BlockSpec(memory_space=pltpu.SMEM),
                  _blk(), _pblk(), _pblk()],
        out_specs=[_blk(), _dinv_blk()],
        out_shape=[jax.ShapeDtypeStruct((N, D), f32),
                   jax.ShapeDtypeStruct((N, 8), f32)],
    )(tau2, x, sxp, degp)


def _stage_bc(h, sp, dinv, w):
    return pl.pallas_call(
        _stage_bc_body,
        grid=_GRID,
        in_specs=[_blk(), _pblk(), _dinv_blk(), _full((D, D))],
        out_specs=_blk(),
        out_shape=jax.ShapeDtypeStruct((N, D), f32),
    )(h, sp, dinv, w)


def _stage_d(g1, g2, s1p, s2p, dinv, wconv, wmlp):
    return pl.pallas_call(
        _stage_d_body,
        grid=_GRID,
        in_specs=[_blk(), _blk(), _pblk(), _pblk(), _dinv_blk(),
                  _full((4 * D, D)), _full((D, OUT))],
        out_specs=pl.BlockSpec((BN, OUT), lambda i: (i, 0)),
        out_shape=jax.ShapeDtypeStruct((N, OUT), f32),
    )(g1, g2, s1p, s2p, dinv, wconv, wmlp)


def kernel(x, edge_index, tau, Wg0, Wg1, Wconv, Wmlp):
    npad = EPAD - E
    src2 = jnp.concatenate(
        [edge_index[0].astype(jnp.int32),
         jnp.zeros((npad,), jnp.int32)]).reshape(NC * NS, ITERS, G)
    dst2 = jnp.concatenate(
        [edge_index[1].astype(jnp.int32),
         jnp.full((npad,), N, jnp.int32)]).reshape(NC * NS, ITERS, G)
    tau2 = jnp.reshape(tau, (1, 1)).astype(f32)

    degp = _deg_count(dst2)
    sxp = _segsum(x, src2, dst2)
    h, dinv = _stage_a(x, sxp, degp, tau2)
    shp = _segsum(h, src2, dst2)
    g1 = _stage_bc(h, shp, dinv, Wg0)
    sg1p = _segsum(g1, src2, dst2)
    g2 = _stage_bc(g1, sg1p, dinv, Wg1)
    sg2p = _segsum(g2, src2, dst2)
    return _stage_d(g1, g2, sg1p, sg2p, dinv, Wconv, Wmlp)


# speedup vs baseline: 1.0012x; 1.0012x over previous
import jax
import jax.numpy as jnp
from jax import lax
from jax.experimental import pallas as pl
from jax.experimental.pallas import tpu as pltpu
from jax.experimental.pallas import tpu_sc as plsc

N = 10000
E = 320000
D = 128
OUT = 64

NC = 2
NS = 16
G = 128
ITERS = 80
CH = 8
NCH = ITERS // CH
EPAD = NC * NS * ITERS * G
NACC = N + 8
ROWS_SUB = 640
ROWS_LAST = N - ROWS_SUB * (NS - 1)

BN = 1000
f32 = jnp.float32


def _zero_fill(zbuf):
    z16 = jnp.zeros((16,), f32)
    for r in range(16):
        for c in range(D // 16):
            zbuf[r, pl.ds(c * 16, 16)] = z16


def _acc_chunks(sid):
    nz = jnp.where(sid < NS - 1, ROWS_SUB // 16, ROWS_LAST // 16)
    nd = jnp.where(sid < NS - 1, ROWS_SUB // 80, ROWS_LAST // 80)
    return sid * ROWS_SUB, nz, nd


def _segsum_body(table, src2, dst2, s_out,
                 srcc0, srcc1, dstc0, dstc1, rows0, rows1, zbuf, acc,
                 gs0, gs1, is0, is1, id0, id1):
    cid = lax.axis_index("c")
    sid = lax.axis_index("s")
    wid = cid * NS + sid
    row0, nz, nd = _acc_chunks(sid)

    srcc = (srcc0, srcc1)
    dstc = (dstc0, dstc1)
    rows = (rows0, rows1)
    gsem = (gs0, gs1)
    isem = (is0, is1)
    idem = (id0, id1)

    def iload(c, p):
        pltpu.async_copy(src2.at[wid, pl.ds(c * CH, CH)], srcc[p], isem[p])
        pltpu.async_copy(dst2.at[wid, pl.ds(c * CH, CH)], dstc[p], idem[p])

    def wait_i(p):
        pltpu.make_async_copy(src2.at[wid, pl.ds(0, CH)], srcc[p],
                              isem[p]).wait()
        pltpu.make_async_copy(dst2.at[wid, pl.ds(0, CH)], dstc[p],
                              idem[p]).wait()

    def gat(p, j, b):
        return pltpu.async_copy(table.at[srcc[p].at[j]], rows[b], gsem[b])

    def sca(p, j, b):
        pltpu.sync_copy(rows[b], acc.at[dstc[p].at[j]], add=True)

    def wait_g(b):
        pltpu.make_async_copy(table.at[srcc[0].at[0]], rows[b],
                              gsem[b]).wait()

    iload(0, 0)
    iload(1, 1)

    _zero_fill(zbuf)
    lax.fori_loop(
        0, nz,
        lambda k, _: (pltpu.sync_copy(zbuf, acc.at[pl.ds(row0 + k * 16, 16)]),
                      0)[1], 0)
    plsc.subcore_barrier()

    wait_i(0)
    gat(0, 0, 0)

    def two_chunks(m, _):
        for half in range(2):
            c = 2 * m + half
            for j in range(CH):
                i = c * CH + j
                b = j % 2
                wait_g(b)
                if j == 0:
                    @pl.when((c > 0) & (c + 1 < NCH))
                    def _():
                        iload(c + 1, 1 - half)
                if j < CH - 1:
                    @pl.when(i < ITERS - 1)
                    def _():
                        gat(half, j + 1, 1 - b)
                else:
                    @pl.when(i < ITERS - 1)
                    def _():
                        wait_i(1 - half)
                        gat(1 - half, 0, 1 - b)
                sca(half, j, b)
        return 0
    lax.fori_loop(0, NCH // 2, two_chunks, 0)

    plsc.subcore_barrier()

    def drain(k, _):
        b = row0 + k * 80
        pltpu.sync_copy(acc.at[pl.ds(b, 80)], s_out.at[cid, pl.ds(b, 80)])
        return 0
    lax.fori_loop(0, nd, drain, 0)


def _deg_body(dst2, deg_out, dstl, ones_v, zbuf, acc, ss0, ss1):
    cid = lax.axis_index("c")
    sid = lax.axis_index("s")
    wid = cid * NS + sid
    row0, nz, nd = _acc_chunks(sid)

    pltpu.sync_copy(dst2.at[wid], dstl)

    _zero_fill(zbuf)
    one16 = jnp.ones((16,), f32)
    for r in range(G):
        for c in range(D // 16):
            ones_v[r, pl.ds(c * 16, 16)] = one16
    lax.fori_loop(
        0, nz,
        lambda k, _: (pltpu.sync_copy(zbuf, acc.at[pl.ds(row0 + k * 16, 16)]),
                      0)[1], 0)
    plsc.subcore_barrier()

    ssem = (ss0, ss1)

    def sca(i, b):
        return pltpu.async_copy(ones_v, acc.at[dstl.at[i]], ssem[b],
                                add=True)

    def wait_s(b):
        pltpu.make_async_copy(ones_v, acc.at[dstl.at[0]], ssem[b]).wait()

    sca(0, 0)
    sca(1, 1)

    def pair(o, _):
        wait_s(0)
        sca(2 * o + 2, 0)
        wait_s(1)
        sca(2 * o + 3, 1)
        return 0
    lax.fori_loop(0, ITERS // 2 - 1, pair, 0)
    wait_s(0)
    wait_s(1)

    plsc.subcore_barrier()

    def drain(k, _):
        b = row0 + k * 80
        pltpu.sync_copy(acc.at[pl.ds(b, 80)], deg_out.at[cid, pl.ds(b, 80)])
        return 0
    lax.fori_loop(0, nd, drain, 0)


_MESH = plsc.VectorSubcoreMesh(core_axis_name="c", subcore_axis_name="s",
                               num_cores=NC, num_subcores=NS)

_segsum = pl.kernel(
    _segsum_body,
    out_type=jax.ShapeDtypeStruct((NC, N, D), f32),
    mesh=_MESH,
    scratch_types=(pltpu.VMEM((CH, G), jnp.int32),
                   pltpu.VMEM((CH, G), jnp.int32),
                   pltpu.VMEM((CH, G), jnp.int32),
                   pltpu.VMEM((CH, G), jnp.int32),
                   pltpu.VMEM((G, D), f32),
                   pltpu.VMEM((G, D), f32),
                   pltpu.VMEM((16, D), f32),
                   pltpu.VMEM_SHARED((NACC, D), f32))
    + (pltpu.SemaphoreType.DMA,) * 6)

_deg_count = pl.kernel(
    _deg_body,
    out_type=jax.ShapeDtypeStruct((NC, N, D), f32),
    mesh=_MESH,
    scratch_types=(pltpu.VMEM((ITERS, G), jnp.int32),
                   pltpu.VMEM((G, D), f32),
                   pltpu.VMEM((16, D), f32),
                   pltpu.VMEM_SHARED((NACC, D), f32),
                   pltpu.SemaphoreType.DMA,
                   pltpu.SemaphoreType.DMA))


def _stage_a_body(tau_ref, x_ref, sxp_ref, degp_ref, h_ref, dinv_ref):
    deg = jnp.maximum(degp_ref[0, :, 0] + degp_ref[1, :, 0], 1.0)
    dinv = (1.0 / deg)[:, None]
    s = sxp_ref[0] + sxp_ref[1]
    xb = x_ref[...]
    h_ref[...] = xb + tau_ref[0, 0] * (s * dinv - xb)
    dinv_ref[...] = jnp.broadcast_to(dinv, (dinv.shape[0], 8))


def _stage_bc_body(h_ref, sp_ref, dinv_ref, w_ref, g_ref):
    dinv = dinv_ref[:, 0][:, None]
    msg = (sp_ref[0] + sp_ref[1]) * dinv - h_ref[...]
    g_ref[...] = jnp.tanh(jnp.dot(msg, w_ref[...],
                                  precision=lax.Precision.HIGHEST))


def _stage_d_body(g1_ref, g2_ref, s1p_ref, s2p_ref, dinv_ref,
                  wconv_ref, wmlp_ref, out_ref):
    dinv = dinv_ref[:, 0][:, None]
    cat = jnp.concatenate(
        [g1_ref[...], g2_ref[...],
         (s1p_ref[0] + s1p_ref[1]) * dinv,
         (s2p_ref[0] + s2p_ref[1]) * dinv], axis=1)
    pre = jax.nn.relu(jnp.dot(cat, wconv_ref[...],
                              precision=lax.Precision.HIGHEST))
    out_ref[...] = jnp.dot(pre, wmlp_ref[...],
                           precision=lax.Precision.HIGHEST)


def _blk():
    return pl.BlockSpec((BN, D), lambda i: (i, 0))


def _pblk():
    return pl.BlockSpec((NC, BN, D), lambda i: (0, i, 0))


def _dinv_blk():
    return pl.BlockSpec((BN, 8), lambda i: (i, 0))


def _full(shape):
    return pl.BlockSpec(shape, lambda i: (0, 0))


_GRID = (N // BN,)


def _stage_a(x, sxp, degp, tau2):
    return pl.pallas_call(
        _stage_a_body,
        grid=_GRID,
        in_specs=[pl.BlockSpec(memory_space=pltpu.SMEM),
                  _blk(), _pblk(), _pblk()],
        out_specs=[_blk(), _dinv_blk()],
        out_shape=[jax.ShapeDtypeStruct((N, D), f32),
                   jax.ShapeDtypeStruct((N, 8), f32)],
    )(tau2, x, sxp, degp)


def _stage_bc(h, sp, dinv, w):
    return pl.pallas_call(
        _stage_bc_body,
        grid=_GRID,
        in_specs=[_blk(), _pblk(), _dinv_blk(), _full((D, D))],
        out_specs=_blk(),
        out_shape=jax.ShapeDtypeStruct((N, D), f32),
    )(h, sp, dinv, w)


def _stage_d(g1, g2, s1p, s2p, dinv, wconv, wmlp):
    return pl.pallas_call(
        _stage_d_body,
        grid=_GRID,
        in_specs=[_blk(), _blk(), _pblk(), _pblk(), _dinv_blk(),
                  _full((4 * D, D)), _full((D, OUT))],
        out_specs=pl.BlockSpec((BN, OUT), lambda i: (i, 0)),
        out_shape=jax.ShapeDtypeStruct((N, OUT), f32),
    )(g1, g2, s1p, s2p, dinv, wconv, wmlp)


def kernel(x, edge_index, tau, Wg0, Wg1, Wconv, Wmlp):
    npad = EPAD - E
    src2 = jnp.concatenate(
        [edge_index[0].astype(jnp.int32),
         jnp.zeros((npad,), jnp.int32)]).reshape(NC * NS, ITERS, G)
    dst2 = jnp.concatenate(
        [edge_index[1].astype(jnp.int32),
         jnp.full((npad,), N, jnp.int32)]).reshape(NC * NS, ITERS, G)
    tau2 = jnp.reshape(tau, (1, 1)).astype(f32)

    degp = _deg_count(dst2)
    sxp = _segsum(x, src2, dst2)
    h, dinv = _stage_a(x, sxp, degp, tau2)
    shp = _segsum(h, src2, dst2)
    g1 = _stage_bc(h, shp, dinv, Wg0)
    sg1p = _segsum(g1, src2, dst2)
    g2 = _stage_bc(g1, sg1p, dinv, Wg1)
    sg2p = _segsum(g2, src2, dst2)
    return _stage_d(g1, g2, sg1p, sg2p, dinv, Wconv, Wmlp)
